# Initial kernel scaffold; baseline (speedup 1.0000x reference)
import jax
import jax.numpy as jnp
from jax.experimental import pallas as pl


def kernel(features, topic, type_mask, edge_metapath_indices_0, edge_metapath_indices_1, edge_metapath_text_indices_0, edge_metapath_text_indices_1, target_idx_0, target_idx_1, node_list_0, node_list_1, attn1, attn2, fc1_w, fc1_b, fc2_w, fc_user_w, fc_user_b):
    raise NotImplementedError("write your pallas kernel here")


# speedup vs baseline: 4.7118x; 4.7118x over previous
import functools

import jax
import jax.numpy as jnp
from jax import lax
from jax.experimental import pallas as pl
from jax.experimental.pallas import tpu as pltpu
from jax.experimental.pallas import tpu_sc as plsc

N_NODES = 10000
NT = 8192
E = 160000
L = 3
D = 128
H = 4
HD = H * D
NSLICE = 64
TPS = NT // NSLICE
NWORK = 32
SPW = NSLICE // NWORK
EP_ROWS = 1280


def _bounds_body(t0_ref, t1_ref, rp0_ref, rp1_ref):
    krow = lax.broadcasted_iota(jnp.int32, (128, 128), 0)

    def one(tref, rpref):
        def body(r, acc):
            row = tref[pl.ds(r, 1), :]
            sid = row >> 7
            return acc + (krow == sid).astype(jnp.float32)

        hist = lax.fori_loop(0, EP_ROWS, body, jnp.zeros((128, 128), jnp.float32))
        hist_row = jnp.sum(hist, axis=1)[None, :]
        s_ids = lax.broadcasted_iota(jnp.int32, (128, 128), 1)
        mask = (s_ids < krow).astype(jnp.float32)
        rp = jnp.sum(mask * hist_row, axis=1, keepdims=True)
        rpref[...] = rp.astype(jnp.int32)

    one(t0_ref, rp0_ref)
    one(t1_ref, rp1_ref)


def _bounds(tgt0, tgt1):
    pad = EP_ROWS * 128 - E
    big = jnp.full((pad,), jnp.int32(1 << 30), jnp.int32)
    t0 = jnp.concatenate([tgt0, big]).reshape(EP_ROWS, 128)
    t1 = jnp.concatenate([tgt1, big]).reshape(EP_ROWS, 128)
    rp0, rp1 = pl.pallas_call(
        _bounds_body,
        out_shape=(
            jax.ShapeDtypeStruct((128, 1), jnp.int32),
            jax.ShapeDtypeStruct((128, 1), jnp.int32),
        ),
    )(t0, t1)
    return rp0.reshape(128), rp1.reshape(128)


def _sc_agg_body(feat, topic, idxf, txt, tgt, nl, a1t, attn2, rp,
                 hp_out,
                 acc_v, den_v, a1_v, ctr_v, a1t_v, attn2_v, nl_v, rp_v,
                 idx_v, txt_v, tgt_v, fr_v, tp_v,
                 sem1, sem2):
    cid = lax.axis_index("c")
    sid = lax.axis_index("s")
    wid = cid * 16 + sid

    pltpu.sync_copy(rp, rp_v)
    pltpu.sync_copy(a1t, a1t_v)
    pltpu.sync_copy(attn2, attn2_v)

    lane = lax.broadcasted_iota(jnp.int32, (16,), 0)
    lane4f = (lane < 4).astype(jnp.float32)
    third = jnp.float32(1.0 / 3.0)
    zero16 = jnp.zeros((16,), jnp.float32)

    def slice_body(r, _):
        k = wid * SPW + r
        t0 = k * TPS

        def zero_body(t, _):
            for j in range(HD // 16):
                acc_v[t, pl.ds(16 * j, 16)] = zero16
            den_v[t, :] = zero16
            return 0

        lax.fori_loop(0, TPS, zero_body, 0)

        pltpu.sync_copy(nl.at[pl.ds(t0, TPS)], nl_v)
        pltpu.async_copy(feat.at[nl_v], ctr_v, sem1).wait()

        def a1_body(t, _):
            s = [None] * H
            for h in range(H):
                v = ctr_v[t, pl.ds(0, 16)] * a1t_v[h, pl.ds(0, 16)]
                for j in range(1, 8):
                    v = v + ctr_v[t, pl.ds(16 * j, 16)] * a1t_v[h, pl.ds(16 * j, 16)]
                s[h] = jnp.sum(v)
            w = jnp.where(lane == 0, s[0],
                          jnp.where(lane == 1, s[1],
                                    jnp.where(lane == 2, s[2], s[3])))
            a1_v[t, :] = w
            return 0

        lax.fori_loop(0, TPS, a1_body, 0)

        rpv = rp_v[pl.ds(k, 16)]
        e0 = rpv[0]
        e1 = rpv[1]
        c0 = e0 // 16
        c1 = (e1 + 15) // 16

        def chunk_body(c, _):
            base = c * 16
            pltpu.sync_copy(idxf.at[pl.ds(base * 3, 48)], idx_v)
            pltpu.sync_copy(txt.at[pl.ds(base, 16)], txt_v)
            pltpu.sync_copy(tgt.at[pl.ds(base, 16)], tgt_v.at[pl.ds(0, 16)])
            cp1 = pltpu.async_copy(feat.at[idx_v], fr_v, sem1)
            cp2 = pltpu.async_copy(topic.at[txt_v], tp_v, sem2)
            cp1.wait()
            cp2.wait()

            def edge_body(e, _):
                b3 = e * 3
                p = [None] * H
                hjs = [None] * 8
                for j in range(8):
                    dj = pl.ds(16 * j, 16)
                    hj = (fr_v[b3, dj] + fr_v[b3 + 1, dj] + fr_v[b3 + 2, dj]) * third \
                        + tp_v[e, dj]
                    hjs[j] = hj
                    for h in range(H):
                        m = hj * attn2_v[h, dj]
                        p[h] = m if j == 0 else p[h] + m
                a2s = [jnp.sum(p[h]) for h in range(H)]

                t = tgt_v[pl.ds(e, 16)][0] - t0
                t_c = jnp.minimum(jnp.maximum(t, 0), TPS - 1)
                a1row = a1_v[t_c, :]
                a2vec = jnp.where(lane == 0, a2s[0],
                                  jnp.where(lane == 1, a2s[1],
                                            jnp.where(lane == 2, a2s[2], a2s[3])))
                a = a1row + a2vec
                a = jnp.maximum(a, jnp.float32(0.01) * a)
                eg = base + e
                vf = ((eg >= e0) & (eg < e1)).astype(jnp.float32)
                g = jnp.exp(a) * lane4f * vf
                plsc.addupdate(den_v.at[t_c, :], g)
                for h in range(H):
                    gspl = zero16 + g[h]
                    for j in range(8):
                        col = 128 * h + 16 * j
                        plsc.addupdate(acc_v.at[t_c, pl.ds(col, 16)], gspl * hjs[j])
                return 0

            lax.fori_loop(0, 16, edge_body, 0)
            return 0

        lax.fori_loop(c0, c1, chunk_body, 0)

        def fin_body(t, _):
            drow = den_v[t, :]
            for h in range(H):
                dspl = zero16 + (drow[h] + jnp.float32(1e-9))
                for j in range(8):
                    col = 128 * h + 16 * j
                    v = acc_v[t, pl.ds(col, 16)] / dspl
                    v = jnp.where(v > 0, v, jnp.exp(v) - jnp.float32(1.0))
                    acc_v[t, pl.ds(col, 16)] = v
            return 0

        lax.fori_loop(0, TPS, fin_body, 0)
        pltpu.sync_copy(acc_v, hp_out.at[pl.ds(t0, TPS), :])
        return 0

    lax.fori_loop(0, SPW, slice_body, 0)


def _sc_agg(feat, topic, idxf, txt, tgt, nl, a1t, attn2, rp):
    mesh = plsc.VectorSubcoreMesh(core_axis_name="c", subcore_axis_name="s")
    f = pl.kernel(
        _sc_agg_body,
        out_type=jax.ShapeDtypeStruct((NT, HD), jnp.float32),
        mesh=mesh,
        compiler_params=pltpu.CompilerParams(needs_layout_passes=False),
        scratch_types=[
            pltpu.VMEM((TPS, HD), jnp.float32),
            pltpu.VMEM((TPS, 16), jnp.float32),
            pltpu.VMEM((TPS, 16), jnp.float32),
            pltpu.VMEM((TPS, D), jnp.float32),
            pltpu.VMEM((H, D), jnp.float32),
            pltpu.VMEM((H, D), jnp.float32),
            pltpu.VMEM((TPS,), jnp.int32),
            pltpu.VMEM((128,), jnp.int32),
            pltpu.VMEM((48,), jnp.int32),
            pltpu.VMEM((16,), jnp.int32),
            pltpu.VMEM((32,), jnp.int32),
            pltpu.VMEM((48, D), jnp.float32),
            pltpu.VMEM((16, D), jnp.float32),
            pltpu.SemaphoreType.DMA,
            pltpu.SemaphoreType.DMA,
        ],
    )
    return f(feat, topic, idxf, txt, tgt, nl, a1t, attn2, rp)


def _scores_body(hp0_ref, hp1_ref, w1_ref, b1_ref, w2_ref, s0_ref, s1_ref):
    i = pl.program_id(0)

    @pl.when(i == 0)
    def _():
        s0_ref[0, 0] = jnp.float32(0.0)
        s1_ref[0, 0] = jnp.float32(0.0)

    w1 = w1_ref[...]
    b1 = b1_ref[...]
    w2 = w2_ref[...]
    z0 = jnp.tanh(jnp.dot(hp0_ref[...], w1, preferred_element_type=jnp.float32) + b1)
    z1 = jnp.tanh(jnp.dot(hp1_ref[...], w1, preferred_element_type=jnp.float32) + b1)
    s0_ref[0, 0] += jnp.sum(z0 * w2)
    s1_ref[0, 0] += jnp.sum(z1 * w2)


def _scores(hp0, hp1, fc1_w, fc1_b, fc2_w):
    BS = 512
    nb = NT // BS
    s0, s1 = pl.pallas_call(
        _scores_body,
        grid=(nb,),
        in_specs=[
            pl.BlockSpec((BS, HD), lambda i: (i, 0)),
            pl.BlockSpec((BS, HD), lambda i: (i, 0)),
            pl.BlockSpec((HD, 128), lambda i: (0, 0)),
            pl.BlockSpec((1, 128), lambda i: (0, 0)),
            pl.BlockSpec((1, 128), lambda i: (0, 0)),
        ],
        out_specs=(
            pl.BlockSpec((1, 1), lambda i: (0, 0), memory_space=pltpu.SMEM),
            pl.BlockSpec((1, 1), lambda i: (0, 0), memory_space=pltpu.SMEM),
        ),
        out_shape=(
            jax.ShapeDtypeStruct((1, 1), jnp.float32),
            jax.ShapeDtypeStruct((1, 1), jnp.float32),
        ),
    )(hp0, hp1, fc1_w, fc1_b.reshape(1, 128), fc2_w.reshape(1, 128))
    return s0, s1


def _combine_body(hp0_ref, hp1_ref, wu_ref, bu_ref, s0_ref, s1_ref,
                  hu_ref, lg_ref, beta_ref):
    i = pl.program_id(0)
    dlt = (s1_ref[0, 0] - s0_ref[0, 0]) / jnp.float32(NT)
    b0 = jnp.float32(1.0) / (jnp.float32(1.0) + jnp.exp(dlt))
    b1 = jnp.float32(1.0) - b0

    @pl.when(i == 0)
    def _():
        col = lax.broadcasted_iota(jnp.int32, (1, 128), 1)
        beta_ref[...] = jnp.where(col == 0, b0, jnp.where(col == 1, b1, 0.0))

    hu = b0 * hp0_ref[...] + b1 * hp1_ref[...]
    hu_ref[...] = hu
    lg_ref[...] = jnp.dot(hu, wu_ref[...], preferred_element_type=jnp.float32) \
        + bu_ref[...]


def _combine(hp0, hp1, fc_user_w, fc_user_b, s0, s1):
    BS = 512
    nb = NT // BS
    return pl.pallas_call(
        _combine_body,
        grid=(nb,),
        in_specs=[
            pl.BlockSpec((BS, HD), lambda i: (i, 0)),
            pl.BlockSpec((BS, HD), lambda i: (i, 0)),
            pl.BlockSpec((HD, D), lambda i: (0, 0)),
            pl.BlockSpec((1, D), lambda i: (0, 0)),
            pl.BlockSpec(memory_space=pltpu.SMEM),
            pl.BlockSpec(memory_space=pltpu.SMEM),
        ],
        out_specs=(
            pl.BlockSpec((BS, HD), lambda i: (i, 0)),
            pl.BlockSpec((BS, D), lambda i: (i, 0)),
            pl.BlockSpec((1, 128), lambda i: (0, 0)),
        ),
        out_shape=(
            jax.ShapeDtypeStruct((NT, HD), jnp.float32),
            jax.ShapeDtypeStruct((NT, D), jnp.float32),
            jax.ShapeDtypeStruct((1, 128), jnp.float32),
        ),
    )(hp0, hp1, fc_user_w, fc_user_b.reshape(1, D), s0, s1)


@jax.jit
def kernel(features, topic, type_mask, edge_metapath_indices_0,
           edge_metapath_indices_1, edge_metapath_text_indices_0,
           edge_metapath_text_indices_1, target_idx_0, target_idx_1,
           node_list_0, node_list_1, attn1, attn2, fc1_w, fc1_b, fc2_w,
           fc_user_w, fc_user_b):
    del type_mask
    i32 = jnp.int32
    idxf0 = edge_metapath_indices_0.astype(i32).reshape(-1)
    idxf1 = edge_metapath_indices_1.astype(i32).reshape(-1)
    txt0 = edge_metapath_text_indices_0.astype(i32)
    txt1 = edge_metapath_text_indices_1.astype(i32)
    tgt0 = target_idx_0.astype(i32)
    tgt1 = target_idx_1.astype(i32)
    nl0 = node_list_0.astype(i32)
    nl1 = node_list_1.astype(i32)
    a1t = attn1.T.reshape(H, D)

    rp0, rp1 = _bounds(tgt0, tgt1)
    hp0 = _sc_agg(features, topic, idxf0, txt0, tgt0, nl0, a1t, attn2, rp0)
    hp1 = _sc_agg(features, topic, idxf1, txt1, tgt1, nl1, a1t, attn2, rp1)
    s0, s1 = _scores(hp0, hp1, fc1_w, fc1_b, fc2_w)
    h_user, logits, beta_mat = _combine(hp0, hp1, fc_user_w, fc_user_b, s0, s1)
    return h_user, logits, beta_mat[0, :2]
